# baseline (speedup 1.0000x reference)
import functools
import math

import jax
import jax.numpy as jnp
import numpy as np
from jax import lax
from jax.experimental import pallas as pl
from jax.experimental.pallas import tpu as pltpu
from jax.experimental.pallas import tpu_sc as plsc

VOCAB = 100000
DIM = 128
B = 1024
L = 200
N = B * L

NUM_CORES = 2
NUM_SUBCORES = 16
NW = NUM_CORES * NUM_SUBCORES
LANES = 16

CHUNK = 128
ROWS_PER_W = N // NW
CHUNKS_PER_W = ROWS_PER_W // CHUNK
GROUPS = CHUNK // LANES
CGRP = DIM // LANES
NRING = 3

NFUSE = 96 * L
SPT = 96 // NUM_SUBCORES
LBLK = 40
NLBLK = L // LBLK


def _make_pe() -> np.ndarray:
    position = np.arange(0, L, dtype=np.float32)[:, None]
    div_term = np.exp(
        np.arange(0, DIM, 2, dtype=np.float32) * (-math.log(10000.0) / DIM))
    pe = np.zeros((L, DIM), dtype=np.float32)
    pe[:, 0::2] = np.sin(position * div_term)
    pe[:, 1::2] = np.cos(position * div_term)
    return pe


_PE = _make_pe()


def _body(tok_hbm, pid_hbm, eid_hbm, gid_hbm,
          table_hbm, pol_hbm, elem_hbm, gend_hbm, pe_hbm,
          out_hbm, fuse_hbm,
          tok_all_v, fidx_all_v, idt_v,
          rows0_v, rows1_v, rows2_v, add0_v, add1_v, add2_v,
          small_v, t6_v, peb_v,
          gsem0, gsem1, gsem2,
          asem0, asem1, asem2, wsem0, wsem1, wsem2):
    sid = lax.axis_index("s")
    wid = sid * NUM_CORES + lax.axis_index("c")

    rows = [rows0_v, rows1_v, rows2_v]
    add = [add0_v, add1_v, add2_v]
    gsem = [gsem0, gsem1, gsem2]
    asem = [asem0, asem1, asem2]
    wsem = [wsem0, wsem1, wsem2]

    pltpu.sync_copy(pol_hbm, small_v.at[pl.ds(0, 4)])
    pltpu.sync_copy(elem_hbm, small_v.at[pl.ds(4, 6)])
    pltpu.sync_copy(gend_hbm, small_v.at[pl.ds(10, 4)])

    for j in range(SPT):
        s_val = sid * SPT + j
        p = s_val // 24
        e = (s_val % 24) // 4 + 4
        g = s_val % 4 + 10
        for cg in range(CGRP):
            sl = pl.ds(cg * LANES, LANES)
            t6_v[j, sl] = small_v[p, sl] + small_v[e, sl] + small_v[g, sl]

    stage = add0_v

    def build_lblk(lb, _):
        pltpu.sync_copy(pe_hbm.at[pl.ds(lb * LBLK, LBLK)],
                        peb_v.at[pl.ds(0, LBLK)])
        for j in range(SPT):
            t6row = [t6_v[j, pl.ds(cg * LANES, LANES)] for cg in range(CGRP)]

            def fill(r, _):
                for cg2 in range(CGRP // 2):
                    sla = pl.ds(cg2 * 2 * LANES, LANES)
                    slb = pl.ds((cg2 * 2 + 1) * LANES, LANES)
                    a = t6row[cg2 * 2] + peb_v[r, sla]
                    b = t6row[cg2 * 2 + 1] + peb_v[r, slb]
                    packed = plsc.pack(a, b, format=plsc.PackFormat.INTERLEAVED)
                    stage[r, pl.ds(cg2 * LANES, LANES)] = plsc.bitcast(
                        packed, jnp.int32)
                return ()

            lax.fori_loop(0, LBLK, fill, (), unroll=2)
            s_val = sid * SPT + j
            pltpu.sync_copy(
                stage.at[pl.ds(0, LBLK)],
                fuse_hbm.at[pl.ds(s_val * L + lb * LBLK, LBLK)])
        return ()

    lax.fori_loop(0, NLBLK, build_lblk, (), unroll=False)
    plsc.subcore_barrier()

    lane_iota = lax.iota(jnp.int32, LANES)
    wslice = pl.ds(wid * ROWS_PER_W, ROWS_PER_W)
    pltpu.sync_copy(tok_hbm.at[wslice], tok_all_v)

    base_w = wid * ROWS_PER_W

    def init_l(g, _):
        sl = pl.ds(g * LANES, LANES)
        fidx_all_v[sl] = lax.rem(base_w + g * LANES + lane_iota, L)
        return ()

    lax.fori_loop(0, ROWS_PER_W // LANES, init_l, (), unroll=4)

    for src, mult in ((pid_hbm, 24 * L), (eid_hbm, 4 * L), (gid_hbm, L)):
        pltpu.sync_copy(src.at[wslice], idt_v)

        def acc(g, _):
            sl = pl.ds(g * LANES, LANES)
            fidx_all_v[sl] = fidx_all_v[sl] + idt_v[sl] * mult
            return ()

        lax.fori_loop(0, ROWS_PER_W // LANES, acc, (), unroll=4)

    def gathers_issue(c, r):
        csl = pl.ds(c * CHUNK, CHUNK)
        pltpu.async_copy(table_hbm.at[tok_all_v.at[csl]], rows[r], gsem[r])
        pltpu.async_copy(fuse_hbm.at[fidx_all_v.at[csl]], add[r], asem[r])

    def gathers_wait(r):
        zsl = pl.ds(0, CHUNK)
        pltpu.make_async_copy(
            table_hbm.at[tok_all_v.at[zsl]], rows[r], gsem[r]).wait()
        pltpu.make_async_copy(
            fuse_hbm.at[fidx_all_v.at[zsl]], add[r], asem[r]).wait()

    def compute(r):
        rp, ap = rows[r], add[r]

        def do_row(row, _):
            for cg2 in range(CGRP // 2):
                v32 = plsc.bitcast(
                    ap[row, pl.ds(cg2 * LANES, LANES)], jnp.bfloat16)
                a, b = plsc.unpack(v32, format=plsc.PackFormat.INTERLEAVED)
                plsc.addupdate(rp.at[row, pl.ds(cg2 * 2 * LANES, LANES)], a)
                plsc.addupdate(
                    rp.at[row, pl.ds((cg2 * 2 + 1) * LANES, LANES)], b)
            return ()

        lax.fori_loop(0, CHUNK, do_row, (), unroll=2)

    def wb_issue(c, r):
        base = (wid * CHUNKS_PER_W + c) * CHUNK
        pltpu.async_copy(rows[r], out_hbm.at[pl.ds(base, CHUNK)], wsem[r])

    def wb_wait(r):
        pltpu.make_async_copy(
            rows[r], out_hbm.at[pl.ds(0, CHUNK)], wsem[r]).wait()

    def wn(cond, fn):
        if isinstance(cond, (bool, np.bool_)):
            if cond:
                fn()
        else:
            pl.when(cond)(fn)

    gathers_issue(0, 0)
    gathers_issue(1, 1)

    def step(i, r):
        r2 = (r + 2) % NRING
        gathers_wait(r)

        def ahead():
            wn(i >= 1, lambda: wb_wait(r2))
            gathers_issue(i + 2, r2)

        wn(i + 2 < CHUNKS_PER_W, ahead)

        compute(r)
        wb_issue(i, r)

    def triple(k, _):
        i0 = 3 * k
        step(i0, 0)
        step(i0 + 1, 1)
        step(i0 + 2, 2)
        return ()

    lax.fori_loop(0, (CHUNKS_PER_W - 2) // NRING, triple, (), unroll=False)
    step(CHUNKS_PER_W - 2, (CHUNKS_PER_W - 2) % NRING)
    step(CHUNKS_PER_W - 1, (CHUNKS_PER_W - 1) % NRING)

    wb_wait(0)
    wb_wait(1)
    wb_wait(2)


def kernel(token_ids, polarity_ids, element_ids, gender_ids,
           token_table, polarity_table, element_table, gender_table):
    tok = token_ids.reshape(N)
    pid = polarity_ids.reshape(N)
    eid = element_ids.reshape(N)
    gid = gender_ids.reshape(N)
    pe = jnp.asarray(_PE)

    mesh = plsc.VectorSubcoreMesh(
        core_axis_name="c", subcore_axis_name="s",
        num_cores=NUM_CORES, num_subcores=NUM_SUBCORES)

    dma = pltpu.SemaphoreType.DMA
    cv = pltpu.VMEM((ROWS_PER_W,), jnp.int32)
    fv = pltpu.VMEM((CHUNK, DIM), jnp.float32)
    bv = pltpu.VMEM((CHUNK, DIM // 2), jnp.int32)
    k = pl.kernel(
        _body,
        out_type=(jax.ShapeDtypeStruct((N, DIM), jnp.float32),
                  jax.ShapeDtypeStruct((NFUSE, DIM // 2), jnp.int32)),
        mesh=mesh,
        compiler_params=pltpu.CompilerParams(
            needs_layout_passes=False, use_tc_tiling_on_sc=False),
        scratch_types=[
            cv, cv, cv,
            fv, fv, fv,
            bv, bv, bv,
            pltpu.VMEM((14, DIM), jnp.float32),
            pltpu.VMEM((SPT, DIM), jnp.float32),
            pltpu.VMEM((LBLK, DIM), jnp.float32),
            dma, dma, dma, dma, dma, dma, dma, dma, dma,
        ],
    )
    out, _ = k(tok, pid, eid, gid,
               token_table, polarity_table, element_table, gender_table, pe)
    return out.reshape(B, L, DIM)


# speedup vs baseline: 1.1433x; 1.1433x over previous
import functools
import math

import jax
import jax.numpy as jnp
import numpy as np
from jax import lax
from jax.experimental import pallas as pl
from jax.experimental.pallas import tpu as pltpu
from jax.experimental.pallas import tpu_sc as plsc

VOCAB = 100000
DIM = 128
B = 1024
L = 200
N = B * L

NUM_CORES = 2
NUM_SUBCORES = 16
NW = NUM_CORES * NUM_SUBCORES
LANES = 16

CHUNK = 128
ROWS_PER_W = N // NW
CHUNKS_PER_W = ROWS_PER_W // CHUNK
GROUPS = CHUNK // LANES
CGRP = DIM // LANES
NRING = 3

NFUSE = 96 * L
SPT = 96 // NUM_SUBCORES
LBLK = 40
NLBLK = L // LBLK


def _make_pe() -> np.ndarray:
    position = np.arange(0, L, dtype=np.float32)[:, None]
    div_term = np.exp(
        np.arange(0, DIM, 2, dtype=np.float32) * (-math.log(10000.0) / DIM))
    pe = np.zeros((L, DIM), dtype=np.float32)
    pe[:, 0::2] = np.sin(position * div_term)
    pe[:, 1::2] = np.cos(position * div_term)
    return pe


_PE = _make_pe()


def _body(tok_hbm, pid_hbm, eid_hbm, gid_hbm,
          table_hbm, pol_hbm, elem_hbm, gend_hbm, pe_hbm,
          out_hbm, fuse_hbm,
          tok0_v, pid0_v, eid0_v, gid0_v, sidx0_v,
          tok1_v, pid1_v, eid1_v, gid1_v, sidx1_v,
          tok2_v, pid2_v, eid2_v, gid2_v, sidx2_v,
          rows0_v, rows1_v, rows2_v, add0_v, add1_v, add2_v,
          small_v, t6_v,
          isem0, isem1, isem2, gsem0, gsem1, gsem2,
          asem0, asem1, asem2, wsem0, wsem1, wsem2):
    sid = lax.axis_index("s")
    wid = sid * NUM_CORES + lax.axis_index("c")

    idxb = [(tok0_v, pid0_v, eid0_v, gid0_v),
            (tok1_v, pid1_v, eid1_v, gid1_v),
            (tok2_v, pid2_v, eid2_v, gid2_v)]
    sidx = [sidx0_v, sidx1_v, sidx2_v]
    rows = [rows0_v, rows1_v, rows2_v]
    add = [add0_v, add1_v, add2_v]
    isem = [isem0, isem1, isem2]
    gsem = [gsem0, gsem1, gsem2]
    asem = [asem0, asem1, asem2]
    wsem = [wsem0, wsem1, wsem2]

    pltpu.sync_copy(pol_hbm, small_v.at[pl.ds(0, 4)])
    pltpu.sync_copy(elem_hbm, small_v.at[pl.ds(4, 6)])
    pltpu.sync_copy(gend_hbm, small_v.at[pl.ds(10, 4)])

    for j in range(SPT):
        s_val = sid * SPT + j
        p = s_val // 24
        e = (s_val % 24) // 4 + 4
        g = s_val % 4 + 10
        for cg in range(CGRP):
            sl = pl.ds(cg * LANES, LANES)
            t6_v[j, sl] = small_v[p, sl] + small_v[e, sl] + small_v[g, sl]

    stage = add0_v
    peb = add1_v

    def build_lblk(lb, _):
        pltpu.sync_copy(pe_hbm.at[pl.ds(lb * LBLK, LBLK)],
                        peb.at[pl.ds(0, LBLK)])
        for j in range(SPT):
            t6row = [t6_v[j, pl.ds(cg * LANES, LANES)] for cg in range(CGRP)]

            def fill(r, _):
                for cg in range(CGRP):
                    sl = pl.ds(cg * LANES, LANES)
                    stage[r, sl] = t6row[cg] + peb[r, sl]
                return ()

            lax.fori_loop(0, LBLK, fill, (), unroll=2)
            s_val = sid * SPT + j
            pltpu.sync_copy(
                stage.at[pl.ds(0, LBLK)],
                fuse_hbm.at[pl.ds(s_val * L + lb * LBLK, LBLK)])
        return ()

    lax.fori_loop(0, NLBLK, build_lblk, (), unroll=False)
    plsc.subcore_barrier()

    lane_iota = lax.iota(jnp.int32, LANES)

    def idx_fetch(c, r):
        crow = wid * CHUNKS_PER_W + c
        pltpu.async_copy(tok_hbm.at[crow], idxb[r][0], isem[r])
        pltpu.async_copy(pid_hbm.at[crow], idxb[r][1], isem[r])
        pltpu.async_copy(eid_hbm.at[crow], idxb[r][2], isem[r])
        pltpu.async_copy(gid_hbm.at[crow], idxb[r][3], isem[r])

    def idx_wait(r):
        for j, src in enumerate((tok_hbm, pid_hbm, eid_hbm, gid_hbm)):
            pltpu.make_async_copy(src.at[0], idxb[r][j], isem[r]).wait()

    def sidx_compute(c, r):
        base = (wid * CHUNKS_PER_W + c) * CHUNK
        _, pb, eb, gb = idxb[r]
        for rg in range(GROUPS):
            sl = pl.ds(rg * LANES, LANES)
            l16 = lax.rem(base + rg * LANES + lane_iota, L)
            sidx[r][sl] = (pb[sl] * 24 + eb[sl] * 4 + gb[sl]) * L + l16

    def gathers_issue(r):
        pltpu.async_copy(table_hbm.at[idxb[r][0]], rows[r], gsem[r])
        pltpu.async_copy(fuse_hbm.at[sidx[r]], add[r], asem[r])

    def gathers_wait(r):
        pltpu.make_async_copy(table_hbm.at[idxb[r][0]], rows[r], gsem[r]).wait()
        pltpu.make_async_copy(fuse_hbm.at[sidx[r]], add[r], asem[r]).wait()

    def compute(r):
        rp, ap = rows[r], add[r]

        def do_row(row, _):
            for cg in range(CGRP):
                sl = pl.ds(cg * LANES, LANES)
                plsc.addupdate(rp.at[row, sl], ap[row, sl])
            return ()

        lax.fori_loop(0, CHUNK, do_row, (), unroll=2)

    def wb_issue(c, r):
        base = (wid * CHUNKS_PER_W + c) * CHUNK
        pltpu.async_copy(rows[r], out_hbm.at[pl.ds(base, CHUNK)], wsem[r])

    def wb_wait(r):
        pltpu.make_async_copy(
            rows[r], out_hbm.at[pl.ds(0, CHUNK)], wsem[r]).wait()

    def wn(cond, fn):
        if isinstance(cond, (bool, np.bool_)):
            if cond:
                fn()
        else:
            pl.when(cond)(fn)

    idx_fetch(0, 0)
    idx_fetch(1, 1)
    idx_fetch(2, 2)
    idx_wait(0)
    sidx_compute(0, 0)
    gathers_issue(0)
    idx_wait(1)
    sidx_compute(1, 1)
    gathers_issue(1)

    def step(i, r):
        r2 = (r + 2) % NRING
        gathers_wait(r)

        wn(i + 3 < CHUNKS_PER_W, lambda: idx_fetch(i + 3, r))

        def ahead():
            idx_wait(r2)
            sidx_compute(i + 2, r2)
            wn(i >= 1, lambda: wb_wait(r2))
            gathers_issue(r2)

        wn(i + 2 < CHUNKS_PER_W, ahead)

        compute(r)
        wb_issue(i, r)

    def triple(k, _):
        i0 = 3 * k
        step(i0, 0)
        step(i0 + 1, 1)
        step(i0 + 2, 2)
        return ()

    lax.fori_loop(0, (CHUNKS_PER_W - 2) // NRING, triple, (), unroll=False)
    step(CHUNKS_PER_W - 2, (CHUNKS_PER_W - 2) % NRING)
    step(CHUNKS_PER_W - 1, (CHUNKS_PER_W - 1) % NRING)

    wb_wait(0)
    wb_wait(1)
    wb_wait(2)


def kernel(token_ids, polarity_ids, element_ids, gender_ids,
           token_table, polarity_table, element_table, gender_table):
    tok = token_ids.reshape(N // CHUNK, CHUNK)
    pid = polarity_ids.reshape(N // CHUNK, CHUNK)
    eid = element_ids.reshape(N // CHUNK, CHUNK)
    gid = gender_ids.reshape(N // CHUNK, CHUNK)
    pe = jnp.asarray(_PE)

    mesh = plsc.VectorSubcoreMesh(
        core_axis_name="c", subcore_axis_name="s",
        num_cores=NUM_CORES, num_subcores=NUM_SUBCORES)

    dma = pltpu.SemaphoreType.DMA
    iv = pltpu.VMEM((CHUNK,), jnp.int32)
    fv = pltpu.VMEM((CHUNK, DIM), jnp.float32)
    k = pl.kernel(
        _body,
        out_type=(jax.ShapeDtypeStruct((N, DIM), jnp.float32),
                  jax.ShapeDtypeStruct((NFUSE, DIM), jnp.float32)),
        mesh=mesh,
        compiler_params=pltpu.CompilerParams(needs_layout_passes=False),
        scratch_types=[
            iv, iv, iv, iv, iv,
            iv, iv, iv, iv, iv,
            iv, iv, iv, iv, iv,
            fv, fv, fv,
            fv, fv, fv,
            pltpu.VMEM((14, DIM), jnp.float32),
            pltpu.VMEM((SPT, DIM), jnp.float32),
            dma, dma, dma, dma, dma, dma,
            dma, dma, dma, dma, dma, dma,
        ],
    )
    out, _ = k(tok, pid, eid, gid,
               token_table, polarity_table, element_table, gender_table, pe)
    return out.reshape(B, L, DIM)
